# baseline (speedup 1.0000x reference)
import functools

import jax
import jax.numpy as jnp
import numpy as np
from jax import lax
from jax.experimental import pallas as pl
from jax.experimental.pallas import tpu as pltpu
from jax.experimental.pallas import tpu_sc as plsc

_N_FIELDS = 26
_VOCAB = 100000
_EMB = 32
_N_CONT = 13
_B = 16384
_H1 = 256
_H2 = 128

_NW = 32
_R = _B * _N_FIELDS
_RPW = _R // _NW
_CH = 512
_NCH = _RPW // _CH
_SCW = 128
_SPC = _CH // _SCW

_QS = 25088
_TROWS = _N_FIELDS * _QS

_NSLOT = 28
_ROWS_OUT = _B * _NSLOT
_BPW = _B // _NW
_XRPW = _BPW * 2
_XHALF = _XRPW // 2

_BM = 2048


def _xform_body(t_ref, o_ref):
    q = pl.program_id(1)
    part = jnp.swapaxes(t_ref[0], 0, 1)
    for qq in range(4):
        @pl.when(q == qq)
        def _():
            o_ref[:, qq * _EMB:(qq + 1) * _EMB] = part


def _xform(tt):
    return pl.pallas_call(
        _xform_body,
        grid=(_N_FIELDS, 4),
        in_specs=[pl.BlockSpec((1, _EMB, _QS), lambda f, q: (f, 0, q))],
        out_specs=pl.BlockSpec((_QS, 128), lambda f, q: (f, 0)),
        out_shape=jax.ShapeDtypeStruct((_TROWS, 128), jnp.float32),
    )(tt)


def _gather_body(table_hbm, idx_hbm, q3_hbm, sidx_hbm, xidx_hbm, xc_hbm,
                 out_hbm, idx_v, q3_v, sidx_v, xidx_v, rows128_v, rows32_v,
                 sem):
    wid = lax.axis_index("s") * 2 + lax.axis_index("c")

    pltpu.sync_copy(idx_hbm.at[wid], idx_v)
    pltpu.sync_copy(q3_hbm.at[wid], q3_v)
    pltpu.sync_copy(sidx_hbm.at[wid], sidx_v)
    pltpu.sync_copy(xidx_hbm.at[wid], xidx_v)

    iota16 = lax.iota(jnp.int32, 16)

    def chunk(c, carry):
        pltpu.async_copy(table_hbm.at[idx_v.at[c]], rows128_v, sem).wait()

        def ext(g, carry2):
            rvec = g * 16 + iota16
            lbase = q3_v[c, pl.ds(g * 16, 16)] * 32
            for m in range(_EMB):
                val = plsc.load_gather(rows128_v, [rvec, lbase + m])
                plsc.store_scatter(rows32_v, [rvec, iota16 * 0 + m], val)
            return carry2

        lax.fori_loop(0, _CH // 16, ext, 0)

        def scat(j, carry2):
            pltpu.async_copy(
                rows32_v.at[pl.ds(j * _SCW, _SCW)],
                out_hbm.at[sidx_v.at[c * _SPC + j]], sem).wait()
            return carry2

        lax.fori_loop(0, _SPC, scat, 0)
        return carry

    lax.fori_loop(0, _NCH, chunk, 0)

    def xhalf(h, carry):
        pltpu.sync_copy(
            xc_hbm.at[pl.ds(wid * _XRPW + h * _XHALF, _XHALF)], rows32_v)

        def xscat(j, carry2):
            pltpu.async_copy(
                rows32_v.at[pl.ds(j * _SCW, _SCW)],
                out_hbm.at[xidx_v.at[h * (_XHALF // _SCW) + j]], sem).wait()
            return carry2

        lax.fori_loop(0, _XHALF // _SCW, xscat, 0)
        return carry

    lax.fori_loop(0, 2, xhalf, 0)


_gather = functools.partial(
    pl.kernel,
    out_type=jax.ShapeDtypeStruct((_ROWS_OUT, _EMB), jnp.float32),
    mesh=plsc.VectorSubcoreMesh(core_axis_name="c", subcore_axis_name="s"),
    compiler_params=pltpu.CompilerParams(use_tc_tiling_on_sc=False,
                                         needs_layout_passes=False),
    scratch_types=[
        pltpu.VMEM((_NCH, _CH), jnp.int32),
        pltpu.VMEM((_NCH, _CH), jnp.int32),
        pltpu.VMEM((_NCH * _SPC, _SCW), jnp.int32),
        pltpu.VMEM((_XRPW // _SCW, _SCW), jnp.int32),
        pltpu.VMEM((_CH, 128), jnp.float32),
        pltpu.VMEM((_CH, _EMB), jnp.float32),
        pltpu.SemaphoreType.DMA,
    ],
)(_gather_body)


_SIDX = np.arange(_R, dtype=np.int32)
_SIDX = (_SIDX // _N_FIELDS) * _NSLOT + _SIDX % _N_FIELDS
_SIDX = _SIDX.reshape(_NW, _NCH * _SPC, _SCW)
_XIDX = np.arange(_B * 2, dtype=np.int32)
_XIDX = (_XIDX // 2) * _NSLOT + _N_FIELDS + _XIDX % 2
_XIDX = _XIDX.reshape(_NW, _XRPW // _SCW, _SCW)


def _mlp_body(e_ref, w1_ref, b1_ref, w2_ref, b2_ref, w3_ref, b3_ref, o_ref):
    h1 = jax.nn.sigmoid(
        jnp.dot(e_ref[...], w1_ref[...], preferred_element_type=jnp.float32)
        + b1_ref[...])
    h2 = jax.nn.sigmoid(
        jnp.dot(h1, w2_ref[...], preferred_element_type=jnp.float32)
        + b2_ref[...])
    o_ref[...] = jax.nn.sigmoid(
        jnp.dot(h2, w3_ref[...], preferred_element_type=jnp.float32)
        + b3_ref[...])


def _mlp(e, w1, b1, w2, b2, w3, b3):
    n_in = _NSLOT * _EMB
    grid = _B // _BM
    return pl.pallas_call(
        _mlp_body,
        grid=(grid,),
        in_specs=[
            pl.BlockSpec((_BM, n_in), lambda i: (i, 0)),
            pl.BlockSpec((n_in, _H1), lambda i: (0, 0)),
            pl.BlockSpec((1, _H1), lambda i: (0, 0)),
            pl.BlockSpec((_H1, _H2), lambda i: (0, 0)),
            pl.BlockSpec((1, _H2), lambda i: (0, 0)),
            pl.BlockSpec((_H2, 1), lambda i: (0, 0)),
            pl.BlockSpec((1, 1), lambda i: (0, 0)),
        ],
        out_specs=pl.BlockSpec((_BM, 1), lambda i: (i, 0)),
        out_shape=jax.ShapeDtypeStruct((_B, 1), jnp.float32),
    )(e, w1, b1, w2, b2, w3, b3)


def kernel(x_cat, x_cont, tables, W1, b1, W2, b2, W3, b3):
    xi = x_cat.astype(jnp.int32)
    q3m = xi // _QS
    offs = (jnp.arange(_N_FIELDS, dtype=jnp.int32) * _QS)[None, :]
    idx128 = (offs + xi - q3m * _QS).reshape(_NW, _NCH, _CH)
    q3 = q3m.reshape(_NW, _NCH, _CH)
    table128 = _xform(jnp.swapaxes(tables, 1, 2))
    xc64 = jnp.pad(x_cont, ((0, 0), (0, 2 * _EMB - _N_CONT))).reshape(
        _B * 2, _EMB)

    e = _gather(table128, idx128, q3, jnp.asarray(_SIDX), jnp.asarray(_XIDX),
                xc64).reshape(_B, _NSLOT * _EMB)

    w1p = jnp.concatenate(
        [W1, jnp.zeros((_NSLOT * _EMB - W1.shape[0], _H1), jnp.float32)], 0)
    return _mlp(e, w1p, b1[None, :], W2, b2[None, :], W3, b3[None, :])


# speedup vs baseline: 1.6369x; 1.6369x over previous
import functools

import jax
import jax.numpy as jnp
import numpy as np
from jax import lax
from jax.experimental import pallas as pl
from jax.experimental.pallas import tpu as pltpu
from jax.experimental.pallas import tpu_sc as plsc

_N_FIELDS = 26
_VOCAB = 100000
_EMB = 32
_N_CONT = 13
_B = 16384
_H1 = 256
_H2 = 128

_NW = 32
_R = _B * _N_FIELDS
_RPW = _R // _NW
_CH = 512
_NCH = _RPW // _CH
_SCW = 128
_SPC = _CH // _SCW

_QS = 25088
_TROWS = _N_FIELDS * _QS

_NSLOT = 28
_ROWS_OUT = _B * _NSLOT
_BPW = _B // _NW
_XRPW = _BPW * 2
_XHALF = _XRPW // 2

_BM = 2048


def _xform_body(t_ref, o_ref):
    q = pl.program_id(1)
    part = jnp.swapaxes(t_ref[0], 0, 1)
    for qq in range(4):
        @pl.when(q == qq)
        def _():
            o_ref[:, qq * _EMB:(qq + 1) * _EMB] = part


def _xform(tt):
    return pl.pallas_call(
        _xform_body,
        grid=(_N_FIELDS, 4),
        in_specs=[pl.BlockSpec((1, _EMB, _QS), lambda f, q: (f, 0, q))],
        out_specs=pl.BlockSpec((_QS, 128), lambda f, q: (f, 0)),
        out_shape=jax.ShapeDtypeStruct((_TROWS, 128), jnp.float32),
    )(tt)


def _gather_body(table_hbm, idx_hbm, sidx_hbm, xidx_hbm, xc_hbm,
                 out_hbm, idx_v, sidx_v, xidx_v, rows32_v, sem):
    wid = lax.axis_index("s") * 2 + lax.axis_index("c")

    pltpu.sync_copy(idx_hbm.at[wid], idx_v)
    pltpu.sync_copy(sidx_hbm.at[wid], sidx_v)
    pltpu.sync_copy(xidx_hbm.at[wid], xidx_v)

    def chunk(c, carry):
        pltpu.async_copy(table_hbm.at[idx_v.at[c]], rows32_v, sem).wait()

        def scat(j, carry2):
            pltpu.async_copy(
                rows32_v.at[pl.ds(j * _SCW, _SCW)],
                out_hbm.at[sidx_v.at[c * _SPC + j]], sem).wait()
            return carry2

        lax.fori_loop(0, _SPC, scat, 0)
        return carry

    lax.fori_loop(0, _NCH, chunk, 0)

    def xhalf(h, carry):
        pltpu.sync_copy(
            xc_hbm.at[pl.ds(wid * _XRPW + h * _XHALF, _XHALF)], rows32_v)

        def xscat(j, carry2):
            pltpu.async_copy(
                rows32_v.at[pl.ds(j * _SCW, _SCW)],
                out_hbm.at[xidx_v.at[h * (_XHALF // _SCW) + j]], sem).wait()
            return carry2

        lax.fori_loop(0, _XHALF // _SCW, xscat, 0)
        return carry

    lax.fori_loop(0, 2, xhalf, 0)


_gather = functools.partial(
    pl.kernel,
    out_type=jax.ShapeDtypeStruct((_ROWS_OUT, _EMB), jnp.float32),
    mesh=plsc.VectorSubcoreMesh(core_axis_name="c", subcore_axis_name="s"),
    compiler_params=pltpu.CompilerParams(use_tc_tiling_on_sc=False,
                                         needs_layout_passes=False),
    scratch_types=[
        pltpu.VMEM((_NCH, _CH), jnp.int32),
        pltpu.VMEM((_NCH * _SPC, _SCW), jnp.int32),
        pltpu.VMEM((_XRPW // _SCW, _SCW), jnp.int32),
        pltpu.VMEM((_CH, _EMB), jnp.float32),
        pltpu.SemaphoreType.DMA,
    ],
)(_gather_body)


_SIDX = np.arange(_R, dtype=np.int32)
_SIDX = (_SIDX // _N_FIELDS) * _NSLOT + _SIDX % _N_FIELDS
_SIDX = _SIDX.reshape(_NW, _NCH * _SPC, _SCW)
_XIDX = np.arange(_B * 2, dtype=np.int32)
_XIDX = (_XIDX // 2) * _NSLOT + _N_FIELDS + _XIDX % 2
_XIDX = _XIDX.reshape(_NW, _XRPW // _SCW, _SCW)


def _mlp_body(e_ref, w1_ref, b1_ref, w2_ref, b2_ref, w3_ref, b3_ref, o_ref):
    h1 = jax.nn.sigmoid(
        jnp.dot(e_ref[...], w1_ref[...], preferred_element_type=jnp.float32)
        + b1_ref[...])
    h2 = jax.nn.sigmoid(
        jnp.dot(h1, w2_ref[...], preferred_element_type=jnp.float32)
        + b2_ref[...])
    o_ref[...] = jax.nn.sigmoid(
        jnp.dot(h2, w3_ref[...], preferred_element_type=jnp.float32)
        + b3_ref[...])


def _mlp(e, w1, b1, w2, b2, w3, b3):
    n_in = _NSLOT * _EMB
    grid = _B // _BM
    return pl.pallas_call(
        _mlp_body,
        grid=(grid,),
        in_specs=[
            pl.BlockSpec((_BM, n_in), lambda i: (i, 0)),
            pl.BlockSpec((n_in, _H1), lambda i: (0, 0)),
            pl.BlockSpec((1, _H1), lambda i: (0, 0)),
            pl.BlockSpec((_H1, _H2), lambda i: (0, 0)),
            pl.BlockSpec((1, _H2), lambda i: (0, 0)),
            pl.BlockSpec((_H2, 1), lambda i: (0, 0)),
            pl.BlockSpec((1, 1), lambda i: (0, 0)),
        ],
        out_specs=pl.BlockSpec((_BM, 1), lambda i: (i, 0)),
        out_shape=jax.ShapeDtypeStruct((_B, 1), jnp.float32),
    )(e, w1, b1, w2, b2, w3, b3)


def kernel(x_cat, x_cont, tables, W1, b1, W2, b2, W3, b3):
    xi = x_cat.astype(jnp.int32)
    q3m = xi // _QS
    offs = (jnp.arange(_N_FIELDS, dtype=jnp.int32) * _QS)[None, :]
    idx32 = ((offs + xi - q3m * _QS) * 4 + q3m).reshape(_NW, _NCH, _CH)
    table32 = _xform(jnp.swapaxes(tables, 1, 2)).reshape(_TROWS * 4, _EMB)
    xc64 = jnp.pad(x_cont, ((0, 0), (0, 2 * _EMB - _N_CONT))).reshape(
        _B * 2, _EMB)

    e = _gather(table32, idx32, jnp.asarray(_SIDX), jnp.asarray(_XIDX),
                xc64).reshape(_B, _NSLOT * _EMB)

    w1p = jnp.concatenate(
        [W1, jnp.zeros((_NSLOT * _EMB - W1.shape[0], _H1), jnp.float32)], 0)
    return _mlp(e, w1p, b1[None, :], W2, b2[None, :], W3, b3[None, :])


# speedup vs baseline: 1.7394x; 1.0626x over previous
import functools

import jax
import jax.numpy as jnp
import numpy as np
from jax import lax
from jax.experimental import pallas as pl
from jax.experimental.pallas import tpu as pltpu
from jax.experimental.pallas import tpu_sc as plsc

_N_FIELDS = 26
_VOCAB = 100000
_EMB = 32
_N_CONT = 13
_B = 16384
_H1 = 256
_H2 = 128

_NW = 32
_R = _B * _N_FIELDS
_RPW = _R // _NW
_CH = 512
_NCH = _RPW // _CH
_SCW = 128
_SPC = _CH // _SCW

_QS = 25088
_TROWS = _N_FIELDS * _QS

_NSLOT = 28
_ROWS_OUT = _B * _NSLOT
_BPW = _B // _NW
_XRPW = _BPW * 2
_XHALF = _XRPW // 2

_BM = 2048


_HQ = _QS // 2


def _xform_body(t0, t1, t2, t3, o_ref):
    h = pl.program_id(1)
    parts = [jnp.swapaxes(t[0], 0, 1) for t in (t0, t1, t2)]
    p3 = jnp.swapaxes(t3[0], 0, 1)
    limit = jnp.where(h == 1, _VOCAB - 3 * _QS - _HQ, _HQ)
    rowid = lax.broadcasted_iota(jnp.int32, (_HQ, _EMB), 0)
    p3 = jnp.where(rowid < limit, p3, 0.0)
    o_ref[...] = jnp.concatenate(parts + [p3], 1)


def _xform(tt):
    def spec(q):
        return pl.BlockSpec((1, _EMB, _HQ), lambda f, h, q=q: (f, 0, 2 * q + h))
    return pl.pallas_call(
        _xform_body,
        grid=(_N_FIELDS, 2),
        in_specs=[spec(0), spec(1), spec(2), spec(3)],
        out_specs=pl.BlockSpec((_HQ, 128), lambda f, h: (2 * f + h, 0)),
        out_shape=jax.ShapeDtypeStruct((_TROWS, 128), jnp.float32),
    )(tt, tt, tt, tt)


def _gather_body(table_hbm, idx_hbm, sidx_hbm, xidx_hbm, xc_hbm,
                 out_hbm, idx_v, sidx_v, xidx_v, rows32_v, sem):
    wid = lax.axis_index("s") * 2 + lax.axis_index("c")

    pltpu.sync_copy(idx_hbm.at[wid], idx_v)
    pltpu.sync_copy(sidx_hbm.at[wid], sidx_v)
    pltpu.sync_copy(xidx_hbm.at[wid], xidx_v)

    def chunk(c, carry):
        pltpu.async_copy(table_hbm.at[idx_v.at[c]], rows32_v, sem).wait()

        def scat(j, carry2):
            pltpu.async_copy(
                rows32_v.at[pl.ds(j * _SCW, _SCW)],
                out_hbm.at[sidx_v.at[c * _SPC + j]], sem).wait()
            return carry2

        lax.fori_loop(0, _SPC, scat, 0)
        return carry

    lax.fori_loop(0, _NCH, chunk, 0)

    def xhalf(h, carry):
        pltpu.sync_copy(
            xc_hbm.at[pl.ds(wid * _XRPW + h * _XHALF, _XHALF)], rows32_v)

        def xscat(j, carry2):
            pltpu.async_copy(
                rows32_v.at[pl.ds(j * _SCW, _SCW)],
                out_hbm.at[xidx_v.at[h * (_XHALF // _SCW) + j]], sem).wait()
            return carry2

        lax.fori_loop(0, _XHALF // _SCW, xscat, 0)
        return carry

    lax.fori_loop(0, 2, xhalf, 0)


_gather = functools.partial(
    pl.kernel,
    out_type=jax.ShapeDtypeStruct((_ROWS_OUT, _EMB), jnp.float32),
    mesh=plsc.VectorSubcoreMesh(core_axis_name="c", subcore_axis_name="s"),
    compiler_params=pltpu.CompilerParams(use_tc_tiling_on_sc=False,
                                         needs_layout_passes=False),
    scratch_types=[
        pltpu.VMEM((_NCH, _CH), jnp.int32),
        pltpu.VMEM((_NCH * _SPC, _SCW), jnp.int32),
        pltpu.VMEM((_XRPW // _SCW, _SCW), jnp.int32),
        pltpu.VMEM((_CH, _EMB), jnp.float32),
        pltpu.SemaphoreType.DMA,
    ],
)(_gather_body)


_SIDX = np.arange(_R, dtype=np.int32)
_SIDX = (_SIDX // _N_FIELDS) * _NSLOT + _SIDX % _N_FIELDS
_SIDX = _SIDX.reshape(_NW, _NCH * _SPC, _SCW)
_XIDX = np.arange(_B * 2, dtype=np.int32)
_XIDX = (_XIDX // 2) * _NSLOT + _N_FIELDS + _XIDX % 2
_XIDX = _XIDX.reshape(_NW, _XRPW // _SCW, _SCW)


def _mlp_body(e_ref, w1_ref, b1_ref, w2_ref, b2_ref, w3_ref, b3_ref, o_ref):
    h1 = jax.nn.sigmoid(
        jnp.dot(e_ref[...], w1_ref[...], preferred_element_type=jnp.float32)
        + b1_ref[...])
    h2 = jax.nn.sigmoid(
        jnp.dot(h1, w2_ref[...], preferred_element_type=jnp.float32)
        + b2_ref[...])
    o_ref[...] = jax.nn.sigmoid(
        jnp.dot(h2, w3_ref[...], preferred_element_type=jnp.float32)
        + b3_ref[...])


def _mlp(e, w1, b1, w2, b2, w3, b3):
    n_in = _NSLOT * _EMB
    grid = _B // _BM
    return pl.pallas_call(
        _mlp_body,
        grid=(grid,),
        in_specs=[
            pl.BlockSpec((_BM, n_in), lambda i: (i, 0)),
            pl.BlockSpec((n_in, _H1), lambda i: (0, 0)),
            pl.BlockSpec((1, _H1), lambda i: (0, 0)),
            pl.BlockSpec((_H1, _H2), lambda i: (0, 0)),
            pl.BlockSpec((1, _H2), lambda i: (0, 0)),
            pl.BlockSpec((_H2, 1), lambda i: (0, 0)),
            pl.BlockSpec((1, 1), lambda i: (0, 0)),
        ],
        out_specs=pl.BlockSpec((_BM, 1), lambda i: (i, 0)),
        out_shape=jax.ShapeDtypeStruct((_B, 1), jnp.float32),
    )(e, w1, b1, w2, b2, w3, b3)


def kernel(x_cat, x_cont, tables, W1, b1, W2, b2, W3, b3):
    xi = x_cat.astype(jnp.int32)
    q3m = xi // _QS
    offs = (jnp.arange(_N_FIELDS, dtype=jnp.int32) * _QS)[None, :]
    idx32 = ((offs + xi - q3m * _QS) * 4 + q3m).reshape(_NW, _NCH, _CH)
    table32 = _xform(jnp.swapaxes(tables, 1, 2)).reshape(_TROWS * 4, _EMB)
    xc64 = jnp.pad(x_cont, ((0, 0), (0, 2 * _EMB - _N_CONT))).reshape(
        _B * 2, _EMB)

    e = _gather(table32, idx32, jnp.asarray(_SIDX), jnp.asarray(_XIDX),
                xc64).reshape(_B, _NSLOT * _EMB)

    w1p = jnp.concatenate(
        [W1, jnp.zeros((_NSLOT * _EMB - W1.shape[0], _H1), jnp.float32)], 0)
    return _mlp(e, w1p, b1[None, :], W2, b2[None, :], W3, b3[None, :])


# speedup vs baseline: 3.3892x; 1.9485x over previous
import functools

import jax
import jax.numpy as jnp
import numpy as np
from jax import lax
from jax.experimental import pallas as pl
from jax.experimental.pallas import tpu as pltpu
from jax.experimental.pallas import tpu_sc as plsc

_N_FIELDS = 26
_VOCAB = 100000
_EMB = 32
_N_CONT = 13
_B = 16384
_H1 = 256
_H2 = 128

_NW = 32
_R = _B * _N_FIELDS
_RPW = _R // _NW
_CH = 512
_NCH = _RPW // _CH
_SCW = 128
_SPC = _CH // _SCW

_QS = 25088
_TROWS = _N_FIELDS * _QS

_NSLOT = 28
_ROWS_OUT = _B * _NSLOT
_BPW = _B // _NW
_XRPW = _BPW * 2
_XHALF = _XRPW // 2

_BM = 2048


_HQ = _QS // 2


def _xform_body(t0, t1, t2, t3, o_ref):
    h = pl.program_id(1)
    limit = jnp.where(h == 1, _VOCAB - 3 * _QS - _HQ, _HQ)
    lane = lax.broadcasted_iota(jnp.int32, (_EMB, _HQ), 1)
    p3 = jnp.where(lane < limit, t3[0], 0.0)
    stacked = jnp.concatenate([t0[0], t1[0], t2[0], p3], 0)
    o_ref[...] = jnp.swapaxes(stacked, 0, 1)


def _xform(tt):
    def spec(q):
        return pl.BlockSpec((1, _EMB, _HQ), lambda f, h, q=q: (f, 0, 2 * q + h))
    return pl.pallas_call(
        _xform_body,
        grid=(_N_FIELDS, 2),
        in_specs=[spec(0), spec(1), spec(2), spec(3)],
        out_specs=pl.BlockSpec((_HQ, 128), lambda f, h: (2 * f + h, 0)),
        out_shape=jax.ShapeDtypeStruct((_TROWS, 128), jnp.float32),
    )(tt, tt, tt, tt)


def _gather_body(table_hbm, idx_hbm, sidx_hbm, xidx_hbm, xc_hbm,
                 out_hbm, idx_v, sidx_v, xidx_v, rows32_v, sem):
    wid = lax.axis_index("s") * 2 + lax.axis_index("c")

    pltpu.sync_copy(idx_hbm.at[wid], idx_v)
    pltpu.sync_copy(sidx_hbm.at[wid], sidx_v)
    pltpu.sync_copy(xidx_hbm.at[wid], xidx_v)

    def chunk(c, carry):
        pltpu.async_copy(table_hbm.at[idx_v.at[c]], rows32_v, sem).wait()

        def scat(j, carry2):
            pltpu.async_copy(
                rows32_v.at[pl.ds(j * _SCW, _SCW)],
                out_hbm.at[sidx_v.at[c * _SPC + j]], sem).wait()
            return carry2

        lax.fori_loop(0, _SPC, scat, 0)
        return carry

    lax.fori_loop(0, _NCH, chunk, 0)

    def xhalf(h, carry):
        pltpu.sync_copy(
            xc_hbm.at[pl.ds(wid * _XRPW + h * _XHALF, _XHALF)], rows32_v)

        def xscat(j, carry2):
            pltpu.async_copy(
                rows32_v.at[pl.ds(j * _SCW, _SCW)],
                out_hbm.at[xidx_v.at[h * (_XHALF // _SCW) + j]], sem).wait()
            return carry2

        lax.fori_loop(0, _XHALF // _SCW, xscat, 0)
        return carry

    lax.fori_loop(0, 2, xhalf, 0)


_gather = functools.partial(
    pl.kernel,
    out_type=jax.ShapeDtypeStruct((_ROWS_OUT, _EMB), jnp.float32),
    mesh=plsc.VectorSubcoreMesh(core_axis_name="c", subcore_axis_name="s"),
    compiler_params=pltpu.CompilerParams(use_tc_tiling_on_sc=False,
                                         needs_layout_passes=False),
    scratch_types=[
        pltpu.VMEM((_NCH, _CH), jnp.int32),
        pltpu.VMEM((_NCH * _SPC, _SCW), jnp.int32),
        pltpu.VMEM((_XRPW // _SCW, _SCW), jnp.int32),
        pltpu.VMEM((_CH, _EMB), jnp.float32),
        pltpu.SemaphoreType.DMA,
    ],
)(_gather_body)


_SIDX = np.arange(_R, dtype=np.int32)
_SIDX = (_SIDX // _N_FIELDS) * _NSLOT + _SIDX % _N_FIELDS
_SIDX = _SIDX.reshape(_NW, _NCH * _SPC, _SCW)
_XIDX = np.arange(_B * 2, dtype=np.int32)
_XIDX = (_XIDX // 2) * _NSLOT + _N_FIELDS + _XIDX % 2
_XIDX = _XIDX.reshape(_NW, _XRPW // _SCW, _SCW)


def _mlp_body(e_ref, w1_ref, b1_ref, w2_ref, b2_ref, w3_ref, b3_ref, o_ref):
    h1 = jax.nn.sigmoid(
        jnp.dot(e_ref[...], w1_ref[...], preferred_element_type=jnp.float32)
        + b1_ref[...])
    h2 = jax.nn.sigmoid(
        jnp.dot(h1, w2_ref[...], preferred_element_type=jnp.float32)
        + b2_ref[...])
    o_ref[...] = jax.nn.sigmoid(
        jnp.dot(h2, w3_ref[...], preferred_element_type=jnp.float32)
        + b3_ref[...])


def _mlp(e, w1, b1, w2, b2, w3, b3):
    n_in = _NSLOT * _EMB
    grid = _B // _BM
    return pl.pallas_call(
        _mlp_body,
        grid=(grid,),
        in_specs=[
            pl.BlockSpec((_BM, n_in), lambda i: (i, 0)),
            pl.BlockSpec((n_in, _H1), lambda i: (0, 0)),
            pl.BlockSpec((1, _H1), lambda i: (0, 0)),
            pl.BlockSpec((_H1, _H2), lambda i: (0, 0)),
            pl.BlockSpec((1, _H2), lambda i: (0, 0)),
            pl.BlockSpec((_H2, 1), lambda i: (0, 0)),
            pl.BlockSpec((1, 1), lambda i: (0, 0)),
        ],
        out_specs=pl.BlockSpec((_BM, 1), lambda i: (i, 0)),
        out_shape=jax.ShapeDtypeStruct((_B, 1), jnp.float32),
    )(e, w1, b1, w2, b2, w3, b3)


def kernel(x_cat, x_cont, tables, W1, b1, W2, b2, W3, b3):
    xi = x_cat.astype(jnp.int32)
    q3m = xi // _QS
    offs = (jnp.arange(_N_FIELDS, dtype=jnp.int32) * _QS)[None, :]
    idx32 = ((offs + xi - q3m * _QS) * 4 + q3m).reshape(_NW, _NCH, _CH)
    table32 = _xform(jnp.swapaxes(tables, 1, 2)).reshape(_TROWS * 4, _EMB)
    xc64 = jnp.pad(x_cont, ((0, 0), (0, 2 * _EMB - _N_CONT))).reshape(
        _B * 2, _EMB)

    e = _gather(table32, idx32, jnp.asarray(_SIDX), jnp.asarray(_XIDX),
                xc64).reshape(_B, _NSLOT * _EMB)

    w1p = jnp.concatenate(
        [W1, jnp.zeros((_NSLOT * _EMB - W1.shape[0], _H1), jnp.float32)], 0)
    return _mlp(e, w1p, b1[None, :], W2, b2[None, :], W3, b3[None, :])
